# baseline (speedup 1.0000x reference)
import functools

import jax
import jax.numpy as jnp
from jax import lax
from jax.experimental import pallas as pl
from jax.experimental.pallas import tpu as pltpu
from jax.experimental.pallas import tpu_sc as plsc

N = 10000
E = 320000
D = 128
L = 16
NC = 2
NS = 16
NW = NC * NS
EPW = E // NW
C = 80
NCHUNK = EPW // C
N_PAD = 10240
RPS = N_PAD // NS


def _sc_segment_sum(features, src3, dst3, w3):
    mesh = plsc.VectorSubcoreMesh(core_axis_name="c", subcore_axis_name="s")

    @functools.partial(
        pl.kernel,
        out_type=jax.ShapeDtypeStruct((NC, N_PAD, D), jnp.float32),
        mesh=mesh,
        scratch_types=[
            pltpu.VMEM((2, C), jnp.int32),
            pltpu.VMEM((2, 1, C), jnp.int32),
            pltpu.VMEM((2, C), jnp.float32),
            pltpu.VMEM((2, C, D), jnp.float32),
            pltpu.VMEM_SHARED((N_PAD, D), jnp.float32),
            [pltpu.SemaphoreType.DMA] * 2,
            [pltpu.SemaphoreType.DMA] * 2,
            [pltpu.SemaphoreType.DMA] * 2,
            [pltpu.SemaphoreType.DMA] * 2,
        ],
    )
    def seg(feat_hbm, src_hbm, dst_hbm, w_hbm, out_hbm,
            srcb, dstb, wbuf, rows, x_sh, esem, dsem, gsem, ssem):
        c = lax.axis_index("c")
        s = lax.axis_index("s")
        wid = c * NS + s

        def issue_e(k, slot):
            pltpu.async_copy(src_hbm.at[wid, k], srcb.at[slot], esem[slot])
            pltpu.async_copy(w_hbm.at[wid, k], wbuf.at[slot], esem[slot])

        def wait_e(slot):
            pltpu.make_async_copy(
                src_hbm.at[0, 0], srcb.at[slot], esem[slot]).wait()
            pltpu.make_async_copy(
                w_hbm.at[0, 0], wbuf.at[slot], esem[slot]).wait()

        def issue_d(k, slot):
            pltpu.async_copy(dst_hbm.at[wid, k], dstb.at[slot, 0], dsem[slot])

        def wait_d(slot):
            pltpu.make_async_copy(
                dst_hbm.at[0, 0], dstb.at[slot, 0], dsem[slot]).wait()

        def issue_g(slot):
            pltpu.async_copy(
                feat_hbm.at[srcb.at[slot]], rows.at[slot], gsem[slot])

        def wait_rows_sem(sem, slot):
            pltpu.make_async_copy(
                feat_hbm.at[pl.ds(0, C)], rows.at[slot], sem).wait()

        def issue_s(slot):
            pltpu.async_copy(
                rows.at[slot], x_sh.at[dstb.at[slot, 0]], ssem[slot],
                add=True)

        def scale(slot):
            for g in range(C // L):
                w16 = wbuf[slot, pl.ds(g * L, L)]
                for j in range(L):
                    e = g * L + j
                    wj = lax.broadcast_in_dim(
                        lax.gather(
                            w16,
                            jnp.full((L, 1), j, jnp.int32),
                            lax.GatherDimensionNumbers(
                                offset_dims=(),
                                collapsed_slice_dims=(0,),
                                start_index_map=(0,),
                            ),
                            (1,),
                            mode=lax.GatherScatterMode.PROMISE_IN_BOUNDS,
                        ),
                        (L,), (0,),
                    )
                    for d8 in range(D // L):
                        rows[slot, e, pl.ds(d8 * L, L)] = (
                            rows[slot, e, pl.ds(d8 * L, L)] * wj)

        def zrow(r, carry):
            for d8 in range(D // L):
                rows[0, r, pl.ds(d8 * L, L)] = jnp.zeros((L,), jnp.float32)
            return carry
        lax.fori_loop(0, C, zrow, 0)
        for k in range(RPS // C):
            pltpu.sync_copy(rows.at[0], x_sh.at[pl.ds(s * RPS + k * C, C)])
        plsc.subcore_barrier()

        issue_e(0, 0)
        issue_e(1, 1)
        issue_d(0, 0)
        issue_d(1, 1)
        wait_e(0)
        issue_g(0)

        NPAIR = NCHUNK // 2

        def pair(jj, carry):
            for b in (0, 1):
                p, q = b, 1 - b
                wait_rows_sem(gsem[p], p)
                wait_e(q)
                issue_g(q)
                scale(p)
                if 2 * (NPAIR - 1) + b + 2 >= NCHUNK:
                    @pl.when(2 * jj + b + 2 < NCHUNK)
                    def _():
                        issue_e(2 * jj + b + 2, p)
                else:
                    issue_e(2 * jj + b + 2, p)
                wait_d(p)
                issue_s(p)
                wait_rows_sem(ssem[p], p)
                if 2 * (NPAIR - 1) + b + 2 >= NCHUNK:
                    @pl.when(2 * jj + b + 2 < NCHUNK)
                    def _():
                        issue_d(2 * jj + b + 2, p)
                else:
                    issue_d(2 * jj + b + 2, p)
            return carry
        lax.fori_loop(0, NPAIR, pair, 0)

        wait_rows_sem(gsem[0], 0)
        scale(0)
        wait_d(0)
        issue_s(0)
        wait_rows_sem(ssem[0], 0)
        plsc.subcore_barrier()

        for k in range(RPS // C):
            r0 = s * RPS + k * C
            slot = k % 2
            pltpu.sync_copy(x_sh.at[pl.ds(r0, C)], rows.at[slot])
            pltpu.sync_copy(rows.at[slot], out_hbm.at[c, pl.ds(r0, C)])

    return seg(features, src3, dst3, w3)


def _tc_combine(features, x0, x1, W1, b1, W2, b2):
    BR = 1000

    def body(f_ref, x0_ref, x1_ref, w1_ref, w2_ref, b1_ref, b2_ref, o_ref):
        x = x0_ref[...] + x1_ref[...]
        f = f_ref[...]
        o_ref[...] = (
            jnp.dot(f + x, w1_ref[...], preferred_element_type=jnp.float32)
            + jnp.dot(x * f, w2_ref[...], preferred_element_type=jnp.float32)
            + b1_ref[...] + b2_ref[...]
        )

    return pl.pallas_call(
        body,
        out_shape=jax.ShapeDtypeStruct((N, D), jnp.float32),
        grid=(N // BR,),
        in_specs=[
            pl.BlockSpec((BR, D), lambda i: (i, 0)),
            pl.BlockSpec((BR, D), lambda i: (i, 0)),
            pl.BlockSpec((BR, D), lambda i: (i, 0)),
            pl.BlockSpec((D, D), lambda i: (0, 0)),
            pl.BlockSpec((D, D), lambda i: (0, 0)),
            pl.BlockSpec((1, D), lambda i: (0, 0)),
            pl.BlockSpec((1, D), lambda i: (0, 0)),
        ],
        out_specs=pl.BlockSpec((BR, D), lambda i: (i, 0)),
    )(features, x0, x1, W1, W2, b1.reshape(1, D), b2.reshape(1, D))


def kernel(features, edge_index, edge_weight, W1, b1, W2, b2):
    src3 = edge_index[0].astype(jnp.int32).reshape(NW, NCHUNK, C)
    dst3 = edge_index[1].astype(jnp.int32).reshape(NW, NCHUNK, C)
    w3 = edge_weight.reshape(NW, NCHUNK, C)
    xp = _sc_segment_sum(features, src3, dst3, w3)
    return _tc_combine(features, xp[0, :N], xp[1, :N], W1, b1, W2, b2)


# speedup vs baseline: 1.0371x; 1.0371x over previous
import functools

import jax
import jax.numpy as jnp
from jax import lax
from jax.experimental import pallas as pl
from jax.experimental.pallas import tpu as pltpu
from jax.experimental.pallas import tpu_sc as plsc

N = 10000
E = 320000
D = 128
L = 16
NC = 2
NS = 16
NW = NC * NS
EPW = E // NW
C = 80
NCHUNK = EPW // C
N_PAD = 10240
RPS = N_PAD // NS


def _sc_segment_sum(features, src3, dst3, w3):
    mesh = plsc.VectorSubcoreMesh(core_axis_name="c", subcore_axis_name="s")

    @functools.partial(
        pl.kernel,
        out_type=jax.ShapeDtypeStruct((NC, N_PAD, D), jnp.float32),
        mesh=mesh,
        scratch_types=[
            pltpu.VMEM((2, C), jnp.int32),
            pltpu.VMEM((2, 1, C), jnp.int32),
            pltpu.VMEM((2, C), jnp.float32),
            pltpu.VMEM((2, C, D), jnp.float32),
            pltpu.VMEM_SHARED((N_PAD, D), jnp.float32),
            [pltpu.SemaphoreType.DMA] * 2,
            [pltpu.SemaphoreType.DMA] * 2,
            [pltpu.SemaphoreType.DMA] * 2,
            [pltpu.SemaphoreType.DMA] * 2,
        ],
    )
    def seg(feat_hbm, src_hbm, dst_hbm, w_hbm, out_hbm,
            srcb, dstb, wbuf, rows, x_sh, esem, dsem, gsem, ssem):
        c = lax.axis_index("c")
        s = lax.axis_index("s")
        wid = c * NS + s

        def issue_e(k, slot):
            pltpu.async_copy(src_hbm.at[wid, k], srcb.at[slot], esem[slot])
            pltpu.async_copy(w_hbm.at[wid, k], wbuf.at[slot], esem[slot])

        def wait_e(slot):
            pltpu.make_async_copy(
                src_hbm.at[0, 0], srcb.at[slot], esem[slot]).wait()
            pltpu.make_async_copy(
                w_hbm.at[0, 0], wbuf.at[slot], esem[slot]).wait()

        def issue_d(k, slot):
            pltpu.async_copy(dst_hbm.at[wid, k], dstb.at[slot, 0], dsem[slot])

        def wait_d(slot):
            pltpu.make_async_copy(
                dst_hbm.at[0, 0], dstb.at[slot, 0], dsem[slot]).wait()

        def issue_g(slot):
            pltpu.async_copy(
                feat_hbm.at[srcb.at[slot]], rows.at[slot], gsem[slot])

        def wait_rows_sem(sem, slot):
            pltpu.make_async_copy(
                feat_hbm.at[pl.ds(0, C)], rows.at[slot], sem).wait()

        def issue_s(slot):
            pltpu.async_copy(
                rows.at[slot], x_sh.at[dstb.at[slot, 0]], ssem[slot],
                add=True)

        def scale(slot):
            for g in range(C // L):
                w16 = wbuf[slot, pl.ds(g * L, L)]
                for j in range(L):
                    e = g * L + j
                    wj = lax.broadcast_in_dim(
                        lax.gather(
                            w16,
                            jnp.full((L, 1), j, jnp.int32),
                            lax.GatherDimensionNumbers(
                                offset_dims=(),
                                collapsed_slice_dims=(0,),
                                start_index_map=(0,),
                            ),
                            (1,),
                            mode=lax.GatherScatterMode.PROMISE_IN_BOUNDS,
                        ),
                        (L,), (0,),
                    )
                    for d8 in range(D // L):
                        rows[slot, e, pl.ds(d8 * L, L)] = (
                            rows[slot, e, pl.ds(d8 * L, L)] * wj)

        def zrow(r, carry):
            for d8 in range(D // L):
                rows[0, r, pl.ds(d8 * L, L)] = jnp.zeros((L,), jnp.float32)
            return carry
        lax.fori_loop(0, C, zrow, 0)
        for k in range(RPS // C):
            pltpu.sync_copy(rows.at[0], x_sh.at[pl.ds(s * RPS + k * C, C)])
        plsc.subcore_barrier()

        issue_e(0, 0)
        issue_e(1, 1)
        issue_d(0, 0)
        issue_d(1, 1)
        wait_e(0)
        issue_g(0)

        NPAIR = NCHUNK // 2

        def pair(jj, carry):
            for b in (0, 1):
                p, q = b, 1 - b
                wait_rows_sem(gsem[p], p)
                wait_e(q)
                issue_g(q)
                scale(p)
                if 2 * (NPAIR - 1) + b + 2 >= NCHUNK:
                    @pl.when(2 * jj + b + 2 < NCHUNK)
                    def _():
                        issue_e(2 * jj + b + 2, p)
                else:
                    issue_e(2 * jj + b + 2, p)
                wait_d(p)
                issue_s(p)
                wait_rows_sem(ssem[p], p)
                if 2 * (NPAIR - 1) + b + 2 >= NCHUNK:
                    @pl.when(2 * jj + b + 2 < NCHUNK)
                    def _():
                        issue_d(2 * jj + b + 2, p)
                else:
                    issue_d(2 * jj + b + 2, p)
            return carry
        lax.fori_loop(0, NPAIR, pair, 0)

        wait_rows_sem(gsem[0], 0)
        scale(0)
        wait_d(0)
        issue_s(0)
        wait_rows_sem(ssem[0], 0)
        plsc.subcore_barrier()

        for k in range(RPS // C):
            r0 = s * RPS + k * C
            slot = k % 2
            pltpu.sync_copy(x_sh.at[pl.ds(r0, C)], rows.at[slot])
            pltpu.sync_copy(rows.at[slot], out_hbm.at[c, pl.ds(r0, C)])

    return seg(features, src3, dst3, w3)


def _tc_combine(features, xp, W1, b1, W2, b2):
    BR = 1000

    def body(f_ref, x0_ref, x1_ref, w1_ref, w2_ref, b1_ref, b2_ref, o_ref):
        x = x0_ref[0] + x1_ref[0]
        f = f_ref[...]
        o_ref[...] = (
            jnp.dot(f + x, w1_ref[...], preferred_element_type=jnp.float32)
            + jnp.dot(x * f, w2_ref[...], preferred_element_type=jnp.float32)
            + b1_ref[...] + b2_ref[...]
        )

    return pl.pallas_call(
        body,
        out_shape=jax.ShapeDtypeStruct((N, D), jnp.float32),
        grid=(N // BR,),
        in_specs=[
            pl.BlockSpec((BR, D), lambda i: (i, 0)),
            pl.BlockSpec((1, BR, D), lambda i: (0, i, 0)),
            pl.BlockSpec((1, BR, D), lambda i: (1, i, 0)),
            pl.BlockSpec((D, D), lambda i: (0, 0)),
            pl.BlockSpec((D, D), lambda i: (0, 0)),
            pl.BlockSpec((1, D), lambda i: (0, 0)),
            pl.BlockSpec((1, D), lambda i: (0, 0)),
        ],
        out_specs=pl.BlockSpec((BR, D), lambda i: (i, 0)),
    )(features, xp, xp, W1, W2, b1.reshape(1, D), b2.reshape(1, D))


def kernel(features, edge_index, edge_weight, W1, b1, W2, b2):
    src3 = edge_index[0].astype(jnp.int32).reshape(NW, NCHUNK, C)
    dst3 = edge_index[1].astype(jnp.int32).reshape(NW, NCHUNK, C)
    w3 = edge_weight.reshape(NW, NCHUNK, C)
    xp = _sc_segment_sum(features, src3, dst3, w3)
    return _tc_combine(features, xp, W1, b1, W2, b2)
